# Initial kernel scaffold; baseline (speedup 1.0000x reference)
import jax
import jax.numpy as jnp
from jax.experimental import pallas as pl


def kernel(x, U, V_w, V_b):
    raise NotImplementedError("write your pallas kernel here")


# speedup vs baseline: 3.9482x; 3.9482x over previous
import functools

import jax
import jax.numpy as jnp
from jax import lax
from jax.experimental import pallas as pl
from jax.experimental.pallas import tpu as pltpu
from jax.experimental.pallas import tpu_sc as plsc

V_N = 1000
D_N = 128
B_N = 16384
C_N = 20

NC = 2
NS = 16
NW = NC * NS
ROWS_W = B_N // NW
BLK = 4
IDX_BLK = BLK * C_N
NBLK = ROWS_W // BLK
LANES = 16


def _sc_pool_body(x_hbm, u_hbm, out_hbm, idx_v, buf_v, acc_v, sem):
    wid = lax.axis_index("s") * NC + lax.axis_index("c")
    row_base = wid * ROWS_W
    pltpu.sync_copy(x_hbm.at[pl.ds(row_base * C_N, ROWS_W * C_N)], idx_v)

    def blk_body(blk, _):
        idx_slice = idx_v.at[pl.ds(blk * IDX_BLK, IDX_BLK)]
        pltpu.async_copy(u_hbm.at[idx_slice], buf_v, sem).wait()
        for r in range(BLK):
            for ch in range(D_N // LANES):
                col = pl.ds(ch * LANES, LANES)
                s = buf_v[r * C_N, col]
                for c in range(1, C_N):
                    s = s + buf_v[r * C_N + c, col]
                acc_v[r, col] = s
        pltpu.sync_copy(acc_v, out_hbm.at[pl.ds(row_base + blk * BLK, BLK)])
        return 0

    lax.fori_loop(0, NBLK, blk_body, 0)


def _sc_pool(x_flat, u):
    mesh = plsc.VectorSubcoreMesh(
        core_axis_name="c", subcore_axis_name="s", num_cores=NC, num_subcores=NS
    )
    fn = pl.kernel(
        _sc_pool_body,
        out_type=jax.ShapeDtypeStruct((B_N, D_N), jnp.float32),
        mesh=mesh,
        scratch_types=[
            pltpu.VMEM((ROWS_W * C_N,), jnp.int32),
            pltpu.VMEM((IDX_BLK, D_N), jnp.float32),
            pltpu.VMEM((BLK, D_N), jnp.float32),
            pltpu.SemaphoreType.DMA,
        ],
    )
    return fn(x_flat, u)


def _tc_proj_body(l1_ref, w_ref, b_ref, o_ref):
    acc = lax.dot_general(
        l1_ref[...], w_ref[...],
        (((1,), (1,)), ((), ())),
        preferred_element_type=jnp.float32,
    )
    o_ref[...] = acc * (1.0 / C_N) + b_ref[...]


def _tc_proj(l1s, v_w, v_b2d):
    tb = 512
    return pl.pallas_call(
        _tc_proj_body,
        grid=(B_N // tb,),
        in_specs=[
            pl.BlockSpec((tb, D_N), lambda i: (i, 0)),
            pl.BlockSpec((V_N, D_N), lambda i: (0, 0)),
            pl.BlockSpec((1, V_N), lambda i: (0, 0)),
        ],
        out_specs=pl.BlockSpec((tb, V_N), lambda i: (i, 0)),
        out_shape=jax.ShapeDtypeStruct((B_N, V_N), jnp.float32),
    )(l1s, v_w, v_b2d)


@jax.jit
def kernel(x, U, V_w, V_b):
    x_flat = x.reshape(-1).astype(jnp.int32)
    l1s = _sc_pool(x_flat, U)
    return _tc_proj(l1s, V_w, V_b.reshape(1, V_N))


# speedup vs baseline: 4.3447x; 1.1004x over previous
import functools

import jax
import jax.numpy as jnp
from jax import lax
from jax.experimental import pallas as pl
from jax.experimental.pallas import tpu as pltpu
from jax.experimental.pallas import tpu_sc as plsc

V_N = 1000
D_N = 128
B_N = 16384
C_N = 20

NC = 2
NS = 16
NW = NC * NS
ROWS_W = B_N // NW
BLK = 4
IDX_BLK = BLK * C_N
NBLK = ROWS_W // BLK
LANES = 16


STAGE_BLKS = 16
STAGE_ROWS = STAGE_BLKS * BLK


def _sum_block(buf_v, acc_v, stage_slot):
    for r in range(BLK):
        for ch in range(D_N // LANES):
            col = pl.ds(ch * LANES, LANES)
            chains = []
            for k in range(4):
                t = buf_v[r * C_N + k, col]
                for c in range(k + 4, C_N, 4):
                    t = t + buf_v[r * C_N + c, col]
                chains.append(t)
            acc_v[stage_slot * BLK + r, col] = (chains[0] + chains[1]) + (
                chains[2] + chains[3]
            )


def _sc_pool_body(x_hbm, u_hbm, out_hbm, idx_v, b0_v, b1_v, acc_v, sem0, sem1):
    wid = lax.axis_index("s") * NC + lax.axis_index("c")
    row_base = wid * ROWS_W
    pltpu.sync_copy(x_hbm.at[pl.ds(row_base * C_N, ROWS_W * C_N)], idx_v)

    def gather(blk, buf, sem):
        idx_slice = idx_v.at[pl.ds(blk * IDX_BLK, IDX_BLK)]
        pltpu.async_copy(u_hbm.at[idx_slice], buf, sem)

    gather(0, b0_v, sem0)
    gather(1, b1_v, sem1)

    def pair_body(i, _):
        for par, (buf, sem) in enumerate(((b0_v, sem0), (b1_v, sem1))):
            blk = 2 * i + par
            pltpu.make_async_copy(
                u_hbm.at[idx_v.at[pl.ds(0, IDX_BLK)]], buf, sem
            ).wait()
            _sum_block(buf, acc_v, blk % STAGE_BLKS)

            @pl.when(blk + 2 < NBLK)
            def _():
                gather(blk + 2, buf, sem)

        @pl.when((2 * i + 2) % STAGE_BLKS == 0)
        def _():
            flush_row = row_base + (2 * i + 2 - STAGE_BLKS) * BLK
            pltpu.sync_copy(acc_v, out_hbm.at[pl.ds(flush_row, STAGE_ROWS)])

        return 0

    lax.fori_loop(0, NBLK // 2, pair_body, 0)


def _sc_pool(x_flat, u):
    mesh = plsc.VectorSubcoreMesh(
        core_axis_name="c", subcore_axis_name="s", num_cores=NC, num_subcores=NS
    )
    fn = pl.kernel(
        _sc_pool_body,
        out_type=jax.ShapeDtypeStruct((B_N, D_N), jnp.float32),
        mesh=mesh,
        scratch_types=[
            pltpu.VMEM((ROWS_W * C_N,), jnp.int32),
            pltpu.VMEM((IDX_BLK, D_N), jnp.float32),
            pltpu.VMEM((IDX_BLK, D_N), jnp.float32),
            pltpu.VMEM((STAGE_ROWS, D_N), jnp.float32),
            pltpu.SemaphoreType.DMA,
            pltpu.SemaphoreType.DMA,
        ],
    )
    return fn(x_flat, u)


def _tc_proj_body(l1_ref, w_ref, b_ref, o_ref):
    acc = lax.dot_general(
        l1_ref[...], w_ref[...],
        (((1,), (1,)), ((), ())),
        preferred_element_type=jnp.float32,
    )
    o_ref[...] = acc * (1.0 / C_N) + b_ref[...]


def _tc_proj(l1s, v_w, v_b2d):
    tb = 512
    return pl.pallas_call(
        _tc_proj_body,
        grid=(B_N // tb,),
        in_specs=[
            pl.BlockSpec((tb, D_N), lambda i: (i, 0)),
            pl.BlockSpec((V_N, D_N), lambda i: (0, 0)),
            pl.BlockSpec((1, V_N), lambda i: (0, 0)),
        ],
        out_specs=pl.BlockSpec((tb, V_N), lambda i: (i, 0)),
        out_shape=jax.ShapeDtypeStruct((B_N, V_N), jnp.float32),
    )(l1s, v_w, v_b2d)


@jax.jit
def kernel(x, U, V_w, V_b):
    x_flat = x.reshape(-1).astype(jnp.int32)
    l1s = _sc_pool(x_flat, U)
    return _tc_proj(l1s, V_w, V_b.reshape(1, V_N))


# speedup vs baseline: 6.4004x; 1.4731x over previous
import functools

import jax
import jax.numpy as jnp
from jax import lax
from jax.experimental import pallas as pl
from jax.experimental.pallas import tpu as pltpu
from jax.experimental.pallas import tpu_sc as plsc

V_N = 1000
D_N = 128
W_N = D_N // 2
B_N = 16384
C_N = 20

NC = 2
NS = 16
NW = NC * NS
ROWS_W = B_N // NW
LANES = 16
GRP = LANES
NGRP = ROWS_W // GRP
FLUSH_G = 16
STAGE_COLS = FLUSH_G * GRP


def _sc_pool_body(xt_hbm, u_hbm, out_hbm, tab_v, idx_v, stage_v, sem):
    wid = lax.axis_index("s") * NC + lax.axis_index("c")
    row_base = pl.multiple_of(wid * ROWS_W, ROWS_W)
    pltpu.sync_copy(u_hbm, tab_v)
    for c in range(C_N):
        pltpu.sync_copy(
            xt_hbm.at[pl.ds(c * B_N + row_base, ROWS_W)],
            idx_v.at[pl.ds(c * ROWS_W, ROWS_W)],
        )

    def grp_body(g, _):
        idxs = [idx_v[pl.ds(c * ROWS_W + g * GRP, GRP)] for c in range(C_N)]
        col = pl.ds((g % FLUSH_G) * GRP, GRP)
        for k in range(W_N):
            kv = jnp.full((LANES,), k * V_N, jnp.int32)
            acc0 = plsc.bitcast(
                plsc.load_gather(tab_v, [idxs[0] + kv]), jnp.bfloat16)
            acc1 = plsc.bitcast(
                plsc.load_gather(tab_v, [idxs[1] + kv]), jnp.bfloat16)
            for c in range(2, C_N, 2):
                acc0 = acc0 + plsc.bitcast(
                    plsc.load_gather(tab_v, [idxs[c] + kv]), jnp.bfloat16)
                acc1 = acc1 + plsc.bitcast(
                    plsc.load_gather(tab_v, [idxs[c + 1] + kv]), jnp.bfloat16)
            lo, hi = plsc.unpack(acc0 + acc1, format=plsc.PackFormat.INTERLEAVED)
            stage_v[2 * k, col] = lo
            stage_v[2 * k + 1, col] = hi

        @pl.when((g + 1) % FLUSH_G == 0)
        def _():
            flush_col = pl.multiple_of(
                row_base + (g + 1 - FLUSH_G) * GRP, STAGE_COLS)
            pltpu.sync_copy(stage_v, out_hbm.at[:, pl.ds(flush_col, STAGE_COLS)])

        return 0

    lax.fori_loop(0, NGRP, grp_body, 0)


def _sc_pool(xt_flat, u_packed_t):
    mesh = plsc.VectorSubcoreMesh(
        core_axis_name="c", subcore_axis_name="s", num_cores=NC, num_subcores=NS
    )
    fn = pl.kernel(
        _sc_pool_body,
        out_type=jax.ShapeDtypeStruct((D_N, B_N), jnp.float32),
        mesh=mesh,
        compiler_params=pltpu.CompilerParams(needs_layout_passes=False),
        scratch_types=[
            pltpu.VMEM((W_N * V_N,), jnp.int32),
            pltpu.VMEM((C_N * ROWS_W,), jnp.int32),
            pltpu.VMEM((D_N, STAGE_COLS), jnp.float32),
            pltpu.SemaphoreType.DMA,
        ],
    )
    return fn(xt_flat, u_packed_t)


def _tc_proj_body(l1_ref, w_ref, b_ref, o_ref):
    acc = lax.dot_general(
        l1_ref[...], w_ref[...],
        (((0,), (1,)), ((), ())),
        preferred_element_type=jnp.float32,
    )
    o_ref[...] = acc * (1.0 / C_N) + b_ref[...]


def _tc_proj(l1t, v_w, v_b2d):
    tb = 2048
    return pl.pallas_call(
        _tc_proj_body,
        grid=(B_N // tb,),
        in_specs=[
            pl.BlockSpec((D_N, tb), lambda i: (0, i)),
            pl.BlockSpec((V_N, D_N), lambda i: (0, 0)),
            pl.BlockSpec((1, V_N), lambda i: (0, 0)),
        ],
        out_specs=pl.BlockSpec((tb, V_N), lambda i: (i, 0)),
        out_shape=jax.ShapeDtypeStruct((B_N, V_N), jnp.float32),
    )(l1t, v_w, v_b2d)


@jax.jit
def kernel(x, U, V_w, V_b):
    xt_flat = x.T.astype(jnp.int32).reshape(-1)
    u_packed_t = lax.bitcast_convert_type(
        U.astype(jnp.bfloat16).reshape(V_N, W_N, 2), jnp.int32
    ).T.reshape(-1)
    l1t = _sc_pool(xt_flat, u_packed_t)
    return _tc_proj(l1t, V_w, V_b.reshape(1, V_N))


# speedup vs baseline: 6.6797x; 1.0436x over previous
import functools

import jax
import jax.numpy as jnp
from jax import lax
from jax.experimental import pallas as pl
from jax.experimental.pallas import tpu as pltpu
from jax.experimental.pallas import tpu_sc as plsc

V_N = 1000
D_N = 128
W_N = D_N // 2
B_N = 16384
C_N = 20

NC = 2
NS = 16
NW = NC * NS
ROWS_W = B_N // NW
LANES = 16
GRP = LANES
NGRP = ROWS_W // GRP
FLUSH_G = 16
STAGE_COLS = FLUSH_G * GRP


def _sc_pool_body(xt_hbm, u_hbm, out_hbm, tab_v, idx_v, stage_v, sem):
    wid = lax.axis_index("s") * NC + lax.axis_index("c")
    row_base = pl.multiple_of(wid * ROWS_W, ROWS_W)
    pltpu.async_copy(u_hbm, tab_v, sem)
    for c in range(C_N):
        pltpu.async_copy(
            xt_hbm.at[pl.ds(c * B_N + row_base, ROWS_W)],
            idx_v.at[pl.ds(c * ROWS_W, ROWS_W)],
            sem,
        )
    pltpu.make_async_copy(u_hbm, tab_v, sem).wait()
    for c in range(C_N):
        pltpu.make_async_copy(
            xt_hbm.at[pl.ds(c * B_N + row_base, ROWS_W)],
            idx_v.at[pl.ds(c * ROWS_W, ROWS_W)],
            sem,
        ).wait()

    def grp_body(g, _):
        idxs = [idx_v[pl.ds(c * ROWS_W + g * GRP, GRP)] for c in range(C_N)]
        col = pl.ds((g % FLUSH_G) * GRP, GRP)
        for k in range(W_N):
            kv = jnp.full((LANES,), k * V_N, jnp.int32)
            acc0 = plsc.bitcast(
                plsc.load_gather(tab_v, [idxs[0] + kv]), jnp.bfloat16)
            acc1 = plsc.bitcast(
                plsc.load_gather(tab_v, [idxs[1] + kv]), jnp.bfloat16)
            for c in range(2, C_N, 2):
                acc0 = acc0 + plsc.bitcast(
                    plsc.load_gather(tab_v, [idxs[c] + kv]), jnp.bfloat16)
                acc1 = acc1 + plsc.bitcast(
                    plsc.load_gather(tab_v, [idxs[c + 1] + kv]), jnp.bfloat16)
            lo, hi = plsc.unpack(acc0 + acc1, format=plsc.PackFormat.INTERLEAVED)
            stage_v[2 * k, col] = lo
            stage_v[2 * k + 1, col] = hi

        @pl.when((g + 1) % FLUSH_G == 0)
        def _():
            flush_col = pl.multiple_of(
                row_base + (g + 1 - FLUSH_G) * GRP, STAGE_COLS)
            pltpu.sync_copy(stage_v, out_hbm.at[:, pl.ds(flush_col, STAGE_COLS)])

        return 0

    lax.fori_loop(0, NGRP, grp_body, 0)


def _sc_pool(xt_flat, u_packed_t):
    mesh = plsc.VectorSubcoreMesh(
        core_axis_name="c", subcore_axis_name="s", num_cores=NC, num_subcores=NS
    )
    fn = pl.kernel(
        _sc_pool_body,
        out_type=jax.ShapeDtypeStruct((D_N, B_N), jnp.float32),
        mesh=mesh,
        compiler_params=pltpu.CompilerParams(needs_layout_passes=False),
        scratch_types=[
            pltpu.VMEM((W_N * V_N,), jnp.int32),
            pltpu.VMEM((C_N * ROWS_W,), jnp.int32),
            pltpu.VMEM((D_N, STAGE_COLS), jnp.float32),
            pltpu.SemaphoreType.DMA,
        ],
    )
    return fn(xt_flat, u_packed_t)


def _tc_proj_body(l1_ref, w_ref, b_ref, o_ref):
    acc = lax.dot_general(
        l1_ref[...], w_ref[...],
        (((0,), (1,)), ((), ())),
        preferred_element_type=jnp.float32,
    )
    o_ref[...] = acc * (1.0 / C_N) + b_ref[...]


def _tc_proj(l1t, v_w, v_b2d):
    tb = 2048
    return pl.pallas_call(
        _tc_proj_body,
        grid=(B_N // tb,),
        in_specs=[
            pl.BlockSpec((D_N, tb), lambda i: (0, i)),
            pl.BlockSpec((V_N, D_N), lambda i: (0, 0)),
            pl.BlockSpec((1, V_N), lambda i: (0, 0)),
        ],
        out_specs=pl.BlockSpec((tb, V_N), lambda i: (i, 0)),
        out_shape=jax.ShapeDtypeStruct((B_N, V_N), jnp.float32),
    )(l1t, v_w, v_b2d)


@jax.jit
def kernel(x, U, V_w, V_b):
    xt_flat = x.T.astype(jnp.int32).reshape(-1)
    u_packed_t = lax.bitcast_convert_type(
        U.astype(jnp.bfloat16).reshape(V_N, W_N, 2), jnp.int32
    ).T.reshape(-1)
    l1t = _sc_pool(xt_flat, u_packed_t)
    return _tc_proj(l1t, V_w, V_b.reshape(1, V_N))
